# baseline (speedup 1.0000x reference)
import functools

import jax
import jax.numpy as jnp
from jax import lax
from jax.experimental import pallas as pl
from jax.experimental.pallas import tpu as pltpu
from jax.experimental.pallas import tpu_sc as plsc

_NLVL, _LAT, _LON = 37, 721, 1440
_NQ = 13
_NCORES = 2
_HB = _LON // _NCORES
_QB = _HB // 6
_NBUF = 8
_LOOKAHEAD = 4


def kernel(field, indices):
    idx = indices.astype(jnp.int32)
    lvls = [idx[i] for i in range(_NQ)]

    @functools.partial(
        pl.kernel,
        out_type=jax.ShapeDtypeStruct((_NQ, _LON, _LAT), jnp.float32),
        mesh=plsc.ScalarSubcoreMesh(axis_name="c", num_cores=_NCORES),
        scratch_types=[
            pltpu.VMEM_SHARED((_NBUF, _QB, _LAT), jnp.float32),
        ] + [pltpu.SemaphoreType.DMA] * (2 * _NBUF),
    )
    def run(field_hbm, out_hbm, buf, *sems):
        gsems, osems = sems[:_NBUF], sems[_NBUF:]
        cid = lax.axis_index("c")
        r0 = pl.multiple_of(cid * _HB, _HB)
        items = [(i, h) for i in range(_NQ) for h in range(6)]
        n = len(items)
        gh, sh = {}, {}

        def g_start(k):
            i, h = items[k]
            b = k % _NBUF
            if k - _NBUF in sh:
                sh[k - _NBUF].wait()
            g = pltpu.make_async_copy(
                field_hbm.at[lvls[i], pl.ds(r0 + h * _QB, _QB)],
                buf.at[b], gsems[b])
            g.start()
            gh[k] = g

        for k in range(_LOOKAHEAD):
            g_start(k)
        for k in range(n):
            i, h = items[k]
            b = k % _NBUF
            gh[k].wait()
            s = pltpu.make_async_copy(
                buf.at[b], out_hbm.at[i, pl.ds(r0 + h * _QB, _QB)], osems[b])
            s.start()
            sh[k] = s
            if k + _LOOKAHEAD < n:
                g_start(k + _LOOKAHEAD)
        for k in range(n - _NBUF, n):
            sh[k].wait()

    field_t = jnp.swapaxes(field, 1, 2)
    out_t = run(field_t)
    return jnp.swapaxes(out_t, 1, 2)


# speedup vs baseline: 1.0636x; 1.0636x over previous
import functools

import jax
import jax.numpy as jnp
from jax import lax
from jax.experimental import pallas as pl
from jax.experimental.pallas import tpu as pltpu
from jax.experimental.pallas import tpu_sc as plsc

_NLVL, _LAT, _LON = 37, 721, 1440
_NQ = 13
_NCORES = 2
_HB = _LON // _NCORES
_QB = _HB // 3
_NBUF = 6
_LOOKAHEAD = 3


def kernel(field, indices):
    idx = indices.astype(jnp.int32)
    lvls = [idx[i] for i in range(_NQ)]

    @functools.partial(
        pl.kernel,
        out_type=jax.ShapeDtypeStruct((_NQ, _LON, _LAT), jnp.float32),
        mesh=plsc.ScalarSubcoreMesh(axis_name="c", num_cores=_NCORES),
        scratch_types=[
            pltpu.VMEM_SHARED((_NBUF, _QB, _LAT), jnp.float32),
        ] + [pltpu.SemaphoreType.DMA] * (2 * _NBUF),
    )
    def run(field_hbm, out_hbm, buf, *sems):
        gsems, osems = sems[:_NBUF], sems[_NBUF:]
        cid = lax.axis_index("c")
        r0 = pl.multiple_of(cid * _HB, _HB)
        items = [(i, h) for i in range(_NQ) for h in range(3)]
        n = len(items)
        gh, sh = {}, {}

        def g_start(k):
            i, h = items[k]
            b = k % _NBUF
            if k - _NBUF in sh:
                sh[k - _NBUF].wait()
            g = pltpu.make_async_copy(
                field_hbm.at[lvls[i], pl.ds(r0 + h * _QB, _QB)],
                buf.at[b], gsems[b])
            g.start()
            gh[k] = g

        for k in range(_LOOKAHEAD):
            g_start(k)
        for k in range(n):
            i, h = items[k]
            b = k % _NBUF
            gh[k].wait()
            s = pltpu.make_async_copy(
                buf.at[b], out_hbm.at[i, pl.ds(r0 + h * _QB, _QB)], osems[b])
            s.start()
            sh[k] = s
            if k + _LOOKAHEAD < n:
                g_start(k + _LOOKAHEAD)
        for k in range(n - _NBUF, n):
            sh[k].wait()

    field_t = jnp.swapaxes(field, 1, 2)
    out_t = run(field_t)
    return jnp.swapaxes(out_t, 1, 2)
